# baseline (speedup 1.0000x reference)
import functools

import jax
import jax.numpy as jnp
from jax import lax
from jax.experimental import pallas as pl
from jax.experimental.pallas import tpu as pltpu
from jax.experimental.pallas import tpu_sc as plsc

_W1 = 1e-06
_W2 = 1.0
_W3 = 1.0
_W4 = 1e-06
_NEG_WEIGHT = 50.0
_GAMMA = 1e-04
_LAMBDA = 2.75

_D = 64
_NNEG = 50
_K = 10
_K_P = 16
_BW = 16
_NC = 2
_NS = 16
_NW = _NC * _NS
_CH = 128


def _merge16(vs, lane):
    for s in range(4):
        sh = 1 << s
        bit = (lane & sh) == 0
        nxt = []
        for m in range(0, len(vs), 2):
            a, b = vs[m], vs[m + 1]
            t = jnp.where(bit, a, b)
            q = jnp.where(bit, b, a)
            nxt.append(t + q[lane ^ sh])
        vs = nxt
    return vs[0]


def _sc_scores(user_table, item_table, bu_t, bi_t, iic_p,
               users, pos_items, neg_flat_h, iin_p):
    B = users.shape[0]
    PB = B // _NW
    NF = PB * _NNEG
    NCH = NF // _CH
    NBF = PB * _K_P
    NBCH = NBF // _CH

    mesh = plsc.VectorSubcoreMesh(core_axis_name="c", subcore_axis_name="s")

    out_type = [
        jax.ShapeDtypeStruct((B,), jnp.float32),
        jax.ShapeDtypeStruct((B * _NNEG,), jnp.float32),
        jax.ShapeDtypeStruct((B * _K_P,), jnp.float32),
        jax.ShapeDtypeStruct((B, _BW), jnp.float32),
        jax.ShapeDtypeStruct((B, _BW), jnp.float32),
        jax.ShapeDtypeStruct((B * _NNEG,), jnp.float32),
        jax.ShapeDtypeStruct((B, _K_P), jnp.float32),
    ]
    scratch_types = [
        pltpu.VMEM((PB,), jnp.int32),
        pltpu.VMEM((PB,), jnp.int32),
        pltpu.VMEM((NF,), jnp.int32),
        pltpu.VMEM((PB, _K_P), jnp.int32),
        pltpu.VMEM((NBF,), jnp.int32),
        pltpu.VMEM((PB, _D), jnp.float32),
        pltpu.VMEM((PB, _D), jnp.float32),
        pltpu.VMEM((PB, _BW), jnp.float32),
        pltpu.VMEM((PB, _BW), jnp.float32),
        pltpu.VMEM((PB, _K_P), jnp.float32),
        pltpu.VMEM((2, _CH, _D), jnp.float32),
        pltpu.VMEM((2, _CH, _D), jnp.float32),
        pltpu.VMEM((2, _CH, _BW), jnp.float32),
        pltpu.VMEM((PB,), jnp.float32),
        pltpu.VMEM((NF,), jnp.float32),
        pltpu.VMEM((NBF,), jnp.float32),
        pltpu.VMEM((NF,), jnp.float32),
        pltpu.SemaphoreType.DMA,
        pltpu.SemaphoreType.DMA,
        pltpu.SemaphoreType.DMA,
    ]

    @functools.partial(
        pl.kernel, out_type=out_type, mesh=mesh, scratch_types=scratch_types,
        compiler_params=pltpu.CompilerParams(use_tc_tiling_on_sc=False))
    def body(ut, it, bu2, bi2, iic, users_h, pos_h, negf_h, iin,
             ps_o, ns_o, ss_o, bu_o, bi_o, bn_o, sim_o,
             users_v, pos_v, neg_flat, nbr_v, nbr_flat, ue_v, pe_v,
             bu_v, bi_v, sim_v, ne_fbuf, nb_fbuf, bnb_fbuf,
             ps_v, ns_flat, ss_flat, bn_flat,
             sem_bulk, sem_a, sem_b):
        wid = lax.axis_index("s") * _NC + lax.axis_index("c")
        base = wid * PB
        lane = lax.iota(jnp.int32, 16)

        pltpu.sync_copy(users_h.at[pl.ds(base, PB)], users_v)
        pltpu.sync_copy(pos_h.at[pl.ds(base, PB)], pos_v)
        pltpu.sync_copy(negf_h.at[pl.ds(base * _NNEG, NF)], neg_flat)

        cps = [
            pltpu.async_copy(ut.at[users_v], ue_v, sem_bulk),
            pltpu.async_copy(it.at[pos_v], pe_v, sem_bulk),
            pltpu.async_copy(bu2.at[users_v], bu_v, sem_bulk),
            pltpu.async_copy(bi2.at[pos_v], bi_v, sem_bulk),
            pltpu.async_copy(iic.at[pos_v], sim_v, sem_bulk),
            pltpu.async_copy(iin.at[pos_v], nbr_v, sem_bulk),
        ]
        for c in cps:
            c.wait()

        def flat_body(e, carry):
            nbr_flat[pl.ds(e * _K_P, _K_P)] = nbr_v[e, pl.ds(0, _K_P)]
            return carry
        lax.fori_loop(0, PB, flat_body, 0)

        def nissue(c, slot, sem):
            idx = neg_flat.at[pl.ds(c * _CH, _CH)]
            pltpu.async_copy(it.at[idx], ne_fbuf.at[slot], sem)
            pltpu.async_copy(bi2.at[idx], bnb_fbuf.at[slot], sem)

        def ndrain(slot, sem):
            pltpu.make_async_copy(it.at[pl.ds(0, _CH)], ne_fbuf.at[slot],
                                  sem).wait()
            pltpu.make_async_copy(bi2.at[pl.ds(0, _CH)], bnb_fbuf.at[slot],
                                  sem).wait()

        def ncompute(c, slot):
            for q in range(_CH // 16):
                f0 = c * _CH + q * 16
                e_lo = f0 // _NNEG
                bnd = (e_lo + 1) * _NNEG - f0
                e_hi = jnp.minimum(e_lo + 1, PB - 1)
                ulo = [ue_v[e_lo, pl.ds(t * 16, 16)] for t in range(4)]
                uhi = [ue_v[e_hi, pl.ds(t * 16, 16)] for t in range(4)]
                vs, bs = [], []
                for r in range(16):
                    row = q * 16 + r
                    cond = r < bnd
                    part = None
                    for t in range(4):
                        u = jnp.where(cond, ulo[t], uhi[t])
                        term = u * ne_fbuf[slot, row, pl.ds(t * 16, 16)]
                        part = term if part is None else part + term
                    vs.append(part)
                    bs.append(bnb_fbuf[slot, row, pl.ds(0, 16)])
                ns_flat[pl.ds(f0, 16)] = _merge16(vs, lane)
                bn_flat[pl.ds(f0, 16)] = _merge16(bs, lane)

        def nloop(h, carry):
            c0 = 2 * h
            nissue(c0 + 1, 1, sem_b)
            ndrain(0, sem_a)
            ncompute(c0, 0)
            nxt = jnp.where(c0 + 2 < NCH, c0 + 2, 0)
            nissue(nxt, 0, sem_a)
            ndrain(1, sem_b)
            ncompute(c0 + 1, 1)
            return carry

        nissue(0, 0, sem_a)

        def pos_body(p, carry):
            vs = []
            for r in range(16):
                elem = p * 16 + r
                part = ue_v[elem, pl.ds(0, 16)] * pe_v[elem, pl.ds(0, 16)]
                for t in range(1, 4):
                    part = part + (ue_v[elem, pl.ds(t * 16, 16)]
                                   * pe_v[elem, pl.ds(t * 16, 16)])
                vs.append(part)
            ps_v[pl.ds(p * 16, 16)] = _merge16(vs, lane)
            return carry
        lax.fori_loop(0, PB // 16, pos_body, 0)

        lax.fori_loop(0, NCH // 2, nloop, 0)
        ndrain(0, sem_a)

        def kissue(c, slot, sem):
            idx = nbr_flat.at[pl.ds(c * _CH, _CH)]
            pltpu.async_copy(it.at[idx], nb_fbuf.at[slot], sem)

        def kdrain(slot, sem):
            pltpu.make_async_copy(it.at[pl.ds(0, _CH)], nb_fbuf.at[slot],
                                  sem).wait()

        def kcompute(c, slot):
            for q in range(_CH // 16):
                elem = c * (_CH // _K_P) + q
                u = [ue_v[elem, pl.ds(t * 16, 16)] for t in range(4)]
                vs = []
                for r in range(16):
                    row = q * 16 + r
                    part = u[0] * nb_fbuf[slot, row, pl.ds(0, 16)]
                    for t in range(1, 4):
                        part = part + u[t] * nb_fbuf[slot, row,
                                                     pl.ds(t * 16, 16)]
                    vs.append(part)
                ss_flat[pl.ds(elem * _K_P, 16)] = _merge16(vs, lane)

        def kloop(h, carry):
            c0 = 2 * h
            kissue(c0 + 1, 1, sem_b)
            kdrain(0, sem_a)
            kcompute(c0, 0)
            nxt = jnp.where(c0 + 2 < NBCH, c0 + 2, 0)
            kissue(nxt, 0, sem_a)
            kdrain(1, sem_b)
            kcompute(c0 + 1, 1)
            return carry

        kissue(0, 0, sem_a)
        lax.fori_loop(0, NBCH // 2, kloop, 0)
        kdrain(0, sem_a)

        pltpu.sync_copy(ps_v, ps_o.at[pl.ds(base, PB)])
        pltpu.sync_copy(ns_flat, ns_o.at[pl.ds(base * _NNEG, NF)])
        pltpu.sync_copy(ss_flat, ss_o.at[pl.ds(base * _K_P, NBF)])
        pltpu.sync_copy(bu_v, bu_o.at[pl.ds(base, PB)])
        pltpu.sync_copy(bi_v, bi_o.at[pl.ds(base, PB)])
        pltpu.sync_copy(bn_flat, bn_o.at[pl.ds(base * _NNEG, NF)])
        pltpu.sync_copy(sim_v, sim_o.at[pl.ds(base, PB)])

    return body(user_table, item_table, bu_t, bi_t, iic_p,
                users, pos_items, neg_flat_h, iin_p)


def _softplus(x):
    return jnp.maximum(x, 0.0) + jnp.log1p(jnp.exp(-jnp.abs(x)))


def _tc_body(nsteps, ut_ref, it_ref, ps_ref, ns_ref, ss_ref, bu_ref, bi_ref,
             bn_ref, sim_ref, out_ref, acc_ref):
    i = pl.program_id(0)

    @pl.when(i == 0)
    def _init():
        acc_ref[0, 0] = 0.0

    x = ut_ref[...]
    y = it_ref[...]
    acc_ref[0, 0] = acc_ref[0, 0] + jnp.sum(x * x) + jnp.sum(y * y)

    @pl.when(i == nsteps - 1)
    def _final():
        B = ps_ref.shape[0]
        bu = bu_ref[...][:, 0:1]
        bi = bi_ref[...][:, 0:1]
        ps = ps_ref[...]
        pos_w = _W1 + _W2 * bu * bi
        pos_loss = jnp.sum(pos_w * _softplus(-ps))

        ns = ns_ref[...]
        bn = bn_ref[...]
        neg_w = _W3 + _W4 * bu * bn
        neg_loss = jnp.sum(neg_w * _softplus(ns)) * (_NEG_WEIGHT / _NNEG)

        ss = ss_ref[...]
        sim = sim_ref[...]
        kcol = lax.broadcasted_iota(jnp.int32, (B, _K_P), 1)
        i_terms = jnp.where(kcol < _K, sim * _softplus(-ss), 0.0)
        loss_i = jnp.sum(i_terms)

        norm = 0.5 * acc_ref[0, 0]
        total = pos_loss + neg_loss + _GAMMA * norm + _LAMBDA * loss_i
        out_ref[...] = jnp.reshape(total, (1, 1))


def _tc_loss(user_table, item_table, ps, ns, ss, bu, bi, bn, sim):
    rows = user_table.shape[0]
    block_rows = 10000
    nsteps = rows // block_rows
    B = ps.shape[0]

    const = lambda i: (0, 0)
    return pl.pallas_call(
        functools.partial(_tc_body, nsteps),
        grid=(nsteps,),
        in_specs=[
            pl.BlockSpec((block_rows, _D), lambda i: (i, 0)),
            pl.BlockSpec((block_rows, _D), lambda i: (i, 0)),
            pl.BlockSpec((B, 1), const),
            pl.BlockSpec((B, _NNEG), const),
            pl.BlockSpec((B, _K_P), const),
            pl.BlockSpec((B, _BW), const),
            pl.BlockSpec((B, _BW), const),
            pl.BlockSpec((B, _NNEG), const),
            pl.BlockSpec((B, _K_P), const),
        ],
        out_specs=pl.BlockSpec((1, 1), const),
        out_shape=jax.ShapeDtypeStruct((1, 1), jnp.float32),
        scratch_shapes=[pltpu.SMEM((1, 1), jnp.float32)],
    )(user_table, item_table, ps, ns, ss, bu, bi, bn, sim)


def kernel(user_table, item_table, beta_uD, beta_iD, ii_constraint,
           users, pos_items, neg_items, ii_neighbor):
    bu_t = jnp.pad(beta_uD[:, None], ((0, 0), (0, _BW - 1)))
    bi_t = jnp.pad(beta_iD[:, None], ((0, 0), (0, _BW - 1)))
    iic_p = jnp.pad(ii_constraint, ((0, 0), (0, _K_P - _K)))
    iin_p = jnp.pad(ii_neighbor, ((0, 0), (0, _K_P - _K)))

    B = users.shape[0]
    ps, ns, ss, bu, bi, bn, sim = _sc_scores(
        user_table, item_table, bu_t, bi_t, iic_p,
        users, pos_items, neg_items.reshape(-1), iin_p)

    out = _tc_loss(user_table, item_table, ps.reshape(B, 1),
                   ns.reshape(B, _NNEG), ss.reshape(B, _K_P),
                   bu, bi, bn.reshape(B, _NNEG), sim)
    return out[0, 0]


# speedup vs baseline: 1.0232x; 1.0232x over previous
import functools

import jax
import jax.numpy as jnp
from jax import lax
from jax.experimental import pallas as pl
from jax.experimental.pallas import tpu as pltpu
from jax.experimental.pallas import tpu_sc as plsc

_W1 = 1e-06
_W2 = 1.0
_W3 = 1.0
_W4 = 1e-06
_NEG_WEIGHT = 50.0
_GAMMA = 1e-04
_LAMBDA = 2.75

_D = 64
_NNEG = 50
_K = 10
_K_P = 16
_BW = 16
_NC = 2
_NS = 16
_NW = _NC * _NS
_CH = 128


def _merge16(vs, lane):
    for s in range(4):
        sh = 1 << s
        bit = (lane & sh) == 0
        nxt = []
        for m in range(0, len(vs), 2):
            a, b = vs[m], vs[m + 1]
            t = jnp.where(bit, a, b)
            q = jnp.where(bit, b, a)
            nxt.append(t + q[lane ^ sh])
        vs = nxt
    return vs[0]


def _sc_scores(user_table, item_table, bu_t, bi_t, iic_p,
               users, pos_items, neg_flat_h, iin_p):
    B = users.shape[0]
    PB = B // _NW
    NF = PB * _NNEG
    NCH = NF // _CH
    NBF = PB * _K_P
    NBCH = NBF // _CH

    mesh = plsc.VectorSubcoreMesh(core_axis_name="c", subcore_axis_name="s")

    out_type = [
        jax.ShapeDtypeStruct((B,), jnp.float32),
        jax.ShapeDtypeStruct((B,), jnp.float32),
        jax.ShapeDtypeStruct((B * _NNEG,), jnp.float32),
        jax.ShapeDtypeStruct((B * _K_P,), jnp.float32),
        jax.ShapeDtypeStruct((B * _NNEG,), jnp.float32),
        jax.ShapeDtypeStruct((B * _K_P,), jnp.float32),
    ]
    scratch_types = [
        pltpu.VMEM((PB,), jnp.int32),
        pltpu.VMEM((PB,), jnp.int32),
        pltpu.VMEM((NF,), jnp.int32),
        pltpu.VMEM((PB, _K_P), jnp.int32),
        pltpu.VMEM((NBF,), jnp.int32),
        pltpu.VMEM((PB, _D), jnp.float32),
        pltpu.VMEM((PB, _D), jnp.float32),
        pltpu.VMEM((PB, _BW), jnp.float32),
        pltpu.VMEM((PB, _BW), jnp.float32),
        pltpu.VMEM((PB, _K_P), jnp.float32),
        pltpu.VMEM((2, _CH, _D), jnp.float32),
        pltpu.VMEM((2, _CH, _D), jnp.float32),
        pltpu.VMEM((2, _CH, _BW), jnp.float32),
        pltpu.VMEM((PB,), jnp.float32),
        pltpu.VMEM((PB,), jnp.float32),
        pltpu.VMEM((NF,), jnp.float32),
        pltpu.VMEM((NBF,), jnp.float32),
        pltpu.VMEM((NF,), jnp.float32),
        pltpu.VMEM((NBF,), jnp.float32),
        pltpu.SemaphoreType.DMA,
        pltpu.SemaphoreType.DMA,
        pltpu.SemaphoreType.DMA,
    ]

    @functools.partial(
        pl.kernel, out_type=out_type, mesh=mesh, scratch_types=scratch_types,
        compiler_params=pltpu.CompilerParams(use_tc_tiling_on_sc=False))
    def body(ut, it, bu2, bi2, iic, users_h, pos_h, negf_h, iin,
             ps_o, pw_o, ns_o, ss_o, bn_o, sim_o,
             users_v, pos_v, neg_flat, nbr_v, nbr_flat, ue_v, pe_v,
             bu_v, bi_v, sim_v, ne_fbuf, nb_fbuf, bnb_fbuf,
             ps_v, pw_v, ns_flat, ss_flat, bn_flat, sim_flat,
             sem_bulk, sem_a, sem_b):
        wid = lax.axis_index("s") * _NC + lax.axis_index("c")
        base = wid * PB
        lane = lax.iota(jnp.int32, 16)

        pltpu.sync_copy(users_h.at[pl.ds(base, PB)], users_v)
        pltpu.sync_copy(pos_h.at[pl.ds(base, PB)], pos_v)
        pltpu.sync_copy(negf_h.at[pl.ds(base * _NNEG, NF)], neg_flat)

        cps = [
            pltpu.async_copy(ut.at[users_v], ue_v, sem_bulk),
            pltpu.async_copy(it.at[pos_v], pe_v, sem_bulk),
            pltpu.async_copy(bu2.at[users_v], bu_v, sem_bulk),
            pltpu.async_copy(bi2.at[pos_v], bi_v, sem_bulk),
            pltpu.async_copy(iic.at[pos_v], sim_v, sem_bulk),
            pltpu.async_copy(iin.at[pos_v], nbr_v, sem_bulk),
        ]
        for c in cps:
            c.wait()

        def flat_body(e, carry):
            nbr_flat[pl.ds(e * _K_P, _K_P)] = nbr_v[e, pl.ds(0, _K_P)]
            sim_flat[pl.ds(e * _K_P, _K_P)] = sim_v[e, pl.ds(0, _K_P)]
            return carry
        lax.fori_loop(0, PB, flat_body, 0)

        def nissue(c, slot, sem):
            idx = neg_flat.at[pl.ds(c * _CH, _CH)]
            pltpu.async_copy(it.at[idx], ne_fbuf.at[slot], sem)
            pltpu.async_copy(bi2.at[idx], bnb_fbuf.at[slot], sem)

        def ndrain(slot, sem):
            pltpu.make_async_copy(it.at[pl.ds(0, _CH)], ne_fbuf.at[slot],
                                  sem).wait()
            pltpu.make_async_copy(bi2.at[pl.ds(0, _CH)], bnb_fbuf.at[slot],
                                  sem).wait()

        def ncompute(c, slot):
            for q in range(_CH // 16):
                f0 = c * _CH + q * 16
                e_lo = f0 // _NNEG
                bnd = (e_lo + 1) * _NNEG - f0
                e_hi = jnp.minimum(e_lo + 1, PB - 1)
                ulo = [ue_v[e_lo, pl.ds(t * 16, 16)] for t in range(4)]
                uhi = [ue_v[e_hi, pl.ds(t * 16, 16)] for t in range(4)]
                blo = bu_v[e_lo, pl.ds(0, 16)][0]
                bhi = bu_v[e_hi, pl.ds(0, 16)][0]
                vs, bs = [], []
                for r in range(16):
                    row = q * 16 + r
                    cond = r < bnd
                    part = None
                    for t in range(4):
                        u = jnp.where(cond, ulo[t], uhi[t])
                        term = u * ne_fbuf[slot, row, pl.ds(t * 16, 16)]
                        part = term if part is None else part + term
                    vs.append(part)
                    bs.append(jnp.where(cond, blo, bhi)
                              * bnb_fbuf[slot, row, pl.ds(0, 16)])
                ns_flat[pl.ds(f0, 16)] = _merge16(vs, lane)
                bn_flat[pl.ds(f0, 16)] = _merge16(bs, lane)

        def nloop(h, carry):
            c0 = 2 * h
            nissue(c0 + 1, 1, sem_b)
            ndrain(0, sem_a)
            ncompute(c0, 0)
            nxt = jnp.where(c0 + 2 < NCH, c0 + 2, 0)
            nissue(nxt, 0, sem_a)
            ndrain(1, sem_b)
            ncompute(c0 + 1, 1)
            return carry

        nissue(0, 0, sem_a)

        def pos_body(p, carry):
            vs, bus, bis = [], [], []
            for r in range(16):
                elem = p * 16 + r
                part = ue_v[elem, pl.ds(0, 16)] * pe_v[elem, pl.ds(0, 16)]
                for t in range(1, 4):
                    part = part + (ue_v[elem, pl.ds(t * 16, 16)]
                                   * pe_v[elem, pl.ds(t * 16, 16)])
                vs.append(part)
                bus.append(bu_v[elem, pl.ds(0, 16)])
                bis.append(bi_v[elem, pl.ds(0, 16)])
            ps_v[pl.ds(p * 16, 16)] = _merge16(vs, lane)
            pw_v[pl.ds(p * 16, 16)] = (
                _W1 + _W2 * _merge16(bus, lane) * _merge16(bis, lane))
            return carry
        lax.fori_loop(0, PB // 16, pos_body, 0)

        lax.fori_loop(0, NCH // 2, nloop, 0)
        ndrain(0, sem_a)

        def kissue(c, slot, sem):
            idx = nbr_flat.at[pl.ds(c * _CH, _CH)]
            pltpu.async_copy(it.at[idx], nb_fbuf.at[slot], sem)

        def kdrain(slot, sem):
            pltpu.make_async_copy(it.at[pl.ds(0, _CH)], nb_fbuf.at[slot],
                                  sem).wait()

        def kcompute(c, slot):
            for q in range(_CH // 16):
                elem = c * (_CH // _K_P) + q
                u = [ue_v[elem, pl.ds(t * 16, 16)] for t in range(4)]
                vs = []
                for r in range(16):
                    row = q * 16 + r
                    part = u[0] * nb_fbuf[slot, row, pl.ds(0, 16)]
                    for t in range(1, 4):
                        part = part + u[t] * nb_fbuf[slot, row,
                                                     pl.ds(t * 16, 16)]
                    vs.append(part)
                ss_flat[pl.ds(elem * _K_P, 16)] = _merge16(vs, lane)

        def kloop(h, carry):
            c0 = 2 * h
            kissue(c0 + 1, 1, sem_b)
            kdrain(0, sem_a)
            kcompute(c0, 0)
            nxt = jnp.where(c0 + 2 < NBCH, c0 + 2, 0)
            kissue(nxt, 0, sem_a)
            kdrain(1, sem_b)
            kcompute(c0 + 1, 1)
            return carry

        kissue(0, 0, sem_a)
        lax.fori_loop(0, NBCH // 2, kloop, 0)
        kdrain(0, sem_a)

        pltpu.sync_copy(ps_v, ps_o.at[pl.ds(base, PB)])
        pltpu.sync_copy(pw_v, pw_o.at[pl.ds(base, PB)])
        pltpu.sync_copy(ns_flat, ns_o.at[pl.ds(base * _NNEG, NF)])
        pltpu.sync_copy(ss_flat, ss_o.at[pl.ds(base * _K_P, NBF)])
        pltpu.sync_copy(bn_flat, bn_o.at[pl.ds(base * _NNEG, NF)])
        pltpu.sync_copy(sim_flat, sim_o.at[pl.ds(base * _K_P, NBF)])

    return body(user_table, item_table, bu_t, bi_t, iic_p,
                users, pos_items, neg_flat_h, iin_p)


def _softplus(x):
    return jnp.maximum(x, 0.0) + jnp.log1p(jnp.exp(-jnp.abs(x)))


def _tc_body(nsteps, ut_ref, it_ref, ps_ref, pw_ref, ns_ref, ss_ref,
             bnw_ref, sim_ref, out_ref, acc_ref):
    i = pl.program_id(0)

    @pl.when(i == 0)
    def _init():
        acc_ref[0, 0] = 0.0

    x = ut_ref[...]
    y = it_ref[...]
    acc_ref[0, 0] = acc_ref[0, 0] + jnp.sum(x * x) + jnp.sum(y * y)

    @pl.when(i == nsteps - 1)
    def _final():
        pos_loss = jnp.sum(pw_ref[...] * _softplus(-ps_ref[...]))

        neg_w = _W3 + _W4 * bnw_ref[...]
        neg_loss = (jnp.sum(neg_w * _softplus(ns_ref[...]))
                    * (_NEG_WEIGHT / _NNEG))

        kcol = lax.broadcasted_iota(jnp.int32, sim_ref.shape, 1) % _K_P
        i_terms = jnp.where(kcol < _K,
                            sim_ref[...] * _softplus(-ss_ref[...]), 0.0)
        loss_i = jnp.sum(i_terms)

        norm = 0.5 * acc_ref[0, 0]
        total = pos_loss + neg_loss + _GAMMA * norm + _LAMBDA * loss_i
        out_ref[...] = jnp.reshape(total, (1, 1))


def _tc_loss(user_table, item_table, ps, pw, ns, ss, bnw, sim):
    rows = user_table.shape[0]
    block_rows = 10000
    nsteps = rows // block_rows

    const = lambda i: (0, 0)
    flat_specs = [pl.BlockSpec(a.shape, const)
                  for a in (ps, pw, ns, ss, bnw, sim)]
    return pl.pallas_call(
        functools.partial(_tc_body, nsteps),
        grid=(nsteps,),
        in_specs=[
            pl.BlockSpec((block_rows, _D), lambda i: (i, 0)),
            pl.BlockSpec((block_rows, _D), lambda i: (i, 0)),
        ] + flat_specs,
        out_specs=pl.BlockSpec((1, 1), const),
        out_shape=jax.ShapeDtypeStruct((1, 1), jnp.float32),
        scratch_shapes=[pltpu.SMEM((1, 1), jnp.float32)],
    )(user_table, item_table, ps, pw, ns, ss, bnw, sim)


def kernel(user_table, item_table, beta_uD, beta_iD, ii_constraint,
           users, pos_items, neg_items, ii_neighbor):
    bu_t = jnp.pad(beta_uD[:, None], ((0, 0), (0, _BW - 1)))
    bi_t = jnp.pad(beta_iD[:, None], ((0, 0), (0, _BW - 1)))
    iic_p = jnp.pad(ii_constraint, ((0, 0), (0, _K_P - _K)))
    iin_p = jnp.pad(ii_neighbor, ((0, 0), (0, _K_P - _K)))

    B = users.shape[0]
    ps, pw, ns, ss, bnw, sim = _sc_scores(
        user_table, item_table, bu_t, bi_t, iic_p,
        users, pos_items, neg_items.reshape(-1), iin_p)

    out = _tc_loss(user_table, item_table,
                   ps.reshape(B // 128, 128), pw.reshape(B // 128, 128),
                   ns.reshape(B * _NNEG // 128, 128),
                   ss.reshape(B * _K_P // 128, 128),
                   bnw.reshape(B * _NNEG // 128, 128),
                   sim.reshape(B * _K_P // 128, 128))
    return out[0, 0]
